# Initial kernel scaffold; baseline (speedup 1.0000x reference)
import jax
import jax.numpy as jnp
from jax.experimental import pallas as pl


def kernel(x, W):
    raise NotImplementedError("write your pallas kernel here")


# speedup vs baseline: 3.9884x; 3.9884x over previous
import functools
import math

import jax
import jax.numpy as jnp
from jax import lax
from jax.experimental import pallas as pl
from jax.experimental.pallas import tpu as pltpu
from jax.experimental.pallas import tpu_sc as plsc

_VOCAB = 100000
_DIM = 64
_B = 4096
_L = 200
_BASE_FREQ = 1e-05

_NC = 2
_NS = 16
_NW = _NC * _NS
_BPW = _B // _NW
_SPLIT = 104


def _pe_table():
    def body(o_ref):
        col = lax.broadcasted_iota(jnp.int32, (_L, _DIM), 1).astype(jnp.float32)
        row = lax.broadcasted_iota(jnp.int32, (_L, _DIM), 0).astype(jnp.float32)
        mult = jnp.exp(row * (2.0 * math.log(_BASE_FREQ) / (_L - 1)))
        o_ref[...] = jnp.sin(col * mult)

    return pl.pallas_call(
        body, out_shape=jax.ShapeDtypeStruct((_L, _DIM), jnp.float32)
    )()


def _sc_body(x_hbm, w_hbm, pe_hbm, out_hbm, idx_v, pe_v, rows0, rows1,
             sem0, sem1):
    wid = lax.axis_index("s") * _NC + lax.axis_index("c")
    base_b = wid * _BPW
    row_base = base_b * _L

    pltpu.sync_copy(x_hbm.at[pl.ds(row_base, _BPW * _L)], idx_v)
    pltpu.sync_copy(pe_hbm, pe_v)

    def fire(j, rows, sem):
        pltpu.async_copy(
            w_hbm.at[idx_v.at[pl.ds(j * _L, _SPLIT)]],
            rows.at[pl.ds(0, _SPLIT), :], sem)
        pltpu.async_copy(
            w_hbm.at[idx_v.at[pl.ds(j * _L + _SPLIT, _L - _SPLIT)]],
            rows.at[pl.ds(_SPLIT, _L - _SPLIT), :], sem)

    def wait_gather(rows, sem):
        pltpu.make_async_copy(w_hbm.at[pl.ds(0, _L), :], rows, sem).wait()

    def process(j, rows):
        def add_row(r, carry):
            for c in range(_DIM // 16):
                plsc.addupdate(rows.at[r, pl.ds(16 * c, 16)],
                               pe_v[r, pl.ds(16 * c, 16)])
            return carry

        lax.fori_loop(0, _L, add_row, 0, unroll=4)
        pltpu.sync_copy(rows, out_hbm.at[pl.ds(row_base + j * _L, _L), :])

    fire(0, rows0, sem0)
    fire(1, rows1, sem1)

    def outer(t, carry):
        for s, (rows, sem) in enumerate(((rows0, sem0), (rows1, sem1))):
            j = 2 * t + s
            wait_gather(rows, sem)
            process(j, rows)

            @pl.when(j + 2 < _BPW)
            def _():
                fire(j + 2, rows, sem)

        return carry

    lax.fori_loop(0, _BPW // 2, outer, 0)


@jax.jit
def kernel(x, W):
    pe = _pe_table()
    x32 = jnp.asarray(x, jnp.int32)

    mesh = plsc.VectorSubcoreMesh(core_axis_name="c", subcore_axis_name="s")
    run = pl.kernel(
        _sc_body,
        out_type=jax.ShapeDtypeStruct((_B * _L, _DIM), jnp.float32),
        mesh=mesh,
        compiler_params=pltpu.CompilerParams(use_tc_tiling_on_sc=False),
        scratch_types=[
            pltpu.VMEM((_BPW * _L,), jnp.int32),
            pltpu.VMEM((_L, _DIM), jnp.float32),
            pltpu.VMEM((_L, _DIM), jnp.float32),
            pltpu.VMEM((_L, _DIM), jnp.float32),
            pltpu.SemaphoreType.DMA,
            pltpu.SemaphoreType.DMA,
        ],
    )
    out = run(x32.reshape(-1), W, pe)
    return out.reshape(_B, _L, _DIM)


# speedup vs baseline: 3.9923x; 1.0010x over previous
import functools
import math

import jax
import jax.numpy as jnp
from jax import lax
from jax.experimental import pallas as pl
from jax.experimental.pallas import tpu as pltpu
from jax.experimental.pallas import tpu_sc as plsc

_VOCAB = 100000
_DIM = 64
_B = 4096
_L = 200
_BASE_FREQ = 1e-05

_NC = 2
_NS = 16
_NW = _NC * _NS
_BPW = _B // _NW
_SPLIT = 104


def _pe_table():
    def body(o_ref):
        col = lax.broadcasted_iota(jnp.int32, (_L, _DIM), 1).astype(jnp.float32)
        row = lax.broadcasted_iota(jnp.int32, (_L, _DIM), 0).astype(jnp.float32)
        mult = jnp.exp(row * (2.0 * math.log(_BASE_FREQ) / (_L - 1)))
        o_ref[...] = jnp.sin(col * mult)

    return pl.pallas_call(
        body, out_shape=jax.ShapeDtypeStruct((_L, _DIM), jnp.float32)
    )()


def _sc_body(x_hbm, w_hbm, pe_hbm, out_hbm, idx_v, pe_v, rows0, rows1,
             sem0, sem1):
    wid = lax.axis_index("s") * _NC + lax.axis_index("c")
    base_b = wid * _BPW
    row_base = base_b * _L

    pltpu.sync_copy(x_hbm.at[pl.ds(base_b, _BPW), :], idx_v)
    pltpu.sync_copy(pe_hbm, pe_v)

    def fire(j, rows, sem):
        pltpu.async_copy(
            w_hbm.at[idx_v.at[j, pl.ds(0, _SPLIT)]],
            rows.at[pl.ds(0, _SPLIT), :], sem)
        pltpu.async_copy(
            w_hbm.at[idx_v.at[j, pl.ds(_SPLIT, _L - _SPLIT)]],
            rows.at[pl.ds(_SPLIT, _L - _SPLIT), :], sem)

    def wait_gather(rows, sem):
        pltpu.make_async_copy(w_hbm.at[pl.ds(0, _L), :], rows, sem).wait()

    def process(j, rows):
        def add_row(r, carry):
            for c in range(_DIM // 16):
                plsc.addupdate(rows.at[r, pl.ds(16 * c, 16)],
                               pe_v[r, pl.ds(16 * c, 16)])
            return carry

        lax.fori_loop(0, _L, add_row, 0, unroll=4)
        pltpu.sync_copy(rows, out_hbm.at[base_b + j])

    fire(0, rows0, sem0)
    fire(1, rows1, sem1)

    def outer(t, carry):
        for s, (rows, sem) in enumerate(((rows0, sem0), (rows1, sem1))):
            j = 2 * t + s
            wait_gather(rows, sem)
            process(j, rows)

            @pl.when(j + 2 < _BPW)
            def _():
                fire(j + 2, rows, sem)

        return carry

    lax.fori_loop(0, _BPW // 2, outer, 0)


@jax.jit
def kernel(x, W):
    pe = _pe_table()
    x32 = jnp.asarray(x, jnp.int32)

    mesh = plsc.VectorSubcoreMesh(core_axis_name="c", subcore_axis_name="s")
    run = pl.kernel(
        _sc_body,
        out_type=jax.ShapeDtypeStruct((_B, _L, _DIM), jnp.float32),
        mesh=mesh,
        compiler_params=pltpu.CompilerParams(use_tc_tiling_on_sc=False),
        scratch_types=[
            pltpu.VMEM((_BPW, _L), jnp.int32),
            pltpu.VMEM((_L, _DIM), jnp.float32),
            pltpu.VMEM((_L, _DIM), jnp.float32),
            pltpu.VMEM((_L, _DIM), jnp.float32),
            pltpu.SemaphoreType.DMA,
            pltpu.SemaphoreType.DMA,
        ],
    )
    return run(x32, W, pe)
